# Initial kernel scaffold; baseline (speedup 1.0000x reference)
import jax
import jax.numpy as jnp
from jax.experimental import pallas as pl


def kernel(x, edge_index, W1, b1, Wmu, bmu, Wls, bls):
    raise NotImplementedError("write your pallas kernel here")


# speedup vs baseline: 10.0220x; 10.0220x over previous
import functools

import jax
import jax.numpy as jnp
from jax import lax
from jax.experimental import pallas as pl
from jax.experimental.pallas import tpu as pltpu
from jax.experimental.pallas import tpu_sc as plsc

N = 10000
E = 320000
NPAD = 10240
CHUNK = 128
NC = 2
NS = 16
NW = NC * NS
RPT = NPAD // NS
WCH = 80
ECH = WCH * NW
EPAD = ECH * CHUNK
PAD_DST = NPAD - 8


def _sc_mesh():
    return plsc.VectorSubcoreMesh(
        core_axis_name="c", subcore_axis_name="s", num_cores=NC, num_subcores=NS
    )


def _fill_rows(rows, val):
    def body(i, _):
        r = i // 8
        col = (i % 8) * 16
        rows[r, pl.ds(col, 16)] = jnp.full((16,), val, jnp.float32)
        return 0

    lax.fori_loop(0, CHUNK * 8, body, 0)


def _zero_acc(rows, acc, s):
    _fill_rows(rows, 0.0)
    for b in range(RPT // CHUNK):
        pltpu.sync_copy(rows, acc.at[pl.ds(s * RPT + b * CHUNK, CHUNK)])


def _copy_out(acc, out_hbm, c, s):
    pltpu.sync_copy(
        acc.at[pl.ds(s * RPT, RPT)], out_hbm.at[c, pl.ds(s * RPT, RPT)]
    )


def _make_deg_kernel():
    @functools.partial(
        pl.kernel,
        mesh=_sc_mesh(),
        out_type=jax.ShapeDtypeStruct((NC, NPAD, 128), jnp.float32),
        scratch_types=[
            pltpu.VMEM((WCH, CHUNK), jnp.int32),
            pltpu.VMEM((CHUNK, 128), jnp.float32),
            pltpu.VMEM_SHARED((NPAD, 128), jnp.float32),
        ],
    )
    def deg_kernel(dst2d_hbm, out_hbm, didx, rows, acc):
        c = lax.axis_index("c")
        s = lax.axis_index("s")
        w = c * NS + s

        _zero_acc(rows, acc, s)
        _fill_rows(rows, 1.0)
        pltpu.sync_copy(dst2d_hbm.at[pl.ds(w * WCH, WCH)], didx)
        plsc.subcore_barrier()

        def chunk_body(j, _):
            pltpu.sync_copy(rows, acc.at[didx.at[j]], add=True)
            return 0

        lax.fori_loop(0, WCH, chunk_body, 0)
        plsc.subcore_barrier()
        _copy_out(acc, out_hbm, c, s)

    return deg_kernel


def _make_prop_kernel():
    HW = WCH // 2

    @functools.partial(
        pl.kernel,
        mesh=_sc_mesh(),
        out_type=jax.ShapeDtypeStruct((NC, NPAD, 128), jnp.float32),
        scratch_types=[
            pltpu.VMEM((HW, CHUNK), jnp.int32),
            pltpu.VMEM((HW, CHUNK), jnp.int32),
            pltpu.VMEM((CHUNK, 128), jnp.float32),
            pltpu.VMEM((CHUNK, 128), jnp.float32),
            pltpu.VMEM_SHARED((NPAD, 128), jnp.float32),
            pltpu.SemaphoreType.DMA,
            pltpu.SemaphoreType.DMA,
        ],
    )
    def prop_kernel(src2d, dst2d, z_hbm, out_hbm, sidx, didx, rows0, rows1, acc, sem0, sem1):
        c = lax.axis_index("c")
        s = lax.axis_index("s")
        w = c * NS + s
        rowsb = (rows0, rows1)
        semb = (sem0, sem1)

        _zero_acc(rows0, acc, s)
        plsc.subcore_barrier()

        def fire(j, b):
            pltpu.async_copy(z_hbm.at[sidx.at[j]], rowsb[b], semb[b])

        def wait(b):
            pltpu.make_async_copy(z_hbm.at[sidx.at[0]], rowsb[b], semb[b]).wait()

        def scat(j, b):
            pltpu.sync_copy(rowsb[b], acc.at[didx.at[j]], add=True)

        for half in range(2):
            pltpu.sync_copy(src2d.at[pl.ds(w * WCH + half * HW, HW)], sidx)
            pltpu.sync_copy(dst2d.at[pl.ds(w * WCH + half * HW, HW)], didx)
            fire(0, 0)

            def pair(j2, _):
                j = 2 * j2
                fire(j + 1, 1)
                wait(0)
                scat(j, 0)

                @pl.when(j + 2 < HW)
                def _():
                    fire(j + 2, 0)

                wait(1)
                scat(j + 1, 1)
                return 0

            lax.fori_loop(0, HW // 2, pair, 0)

        plsc.subcore_barrier()
        _copy_out(acc, out_hbm, c, s)

    return prop_kernel


_deg_kernel = _make_deg_kernel()
_prop128 = _make_prop_kernel()

BR = 2000


def _dinv_of(degp_ref):
    deg = degp_ref[0, :, 0:1] + degp_ref[1, :, 0:1] + 1.0
    return lax.rsqrt(deg)


def _tc_scale(degp, x):
    def body(degp_ref, x_ref, z_ref, dinv_ref):
        dinv = _dinv_of(degp_ref)
        z_ref[...] = x_ref[...] * dinv
        dinv_ref[...] = dinv

    return pl.pallas_call(
        body,
        grid=(N // BR,),
        in_specs=[
            pl.BlockSpec((2, BR, 128), lambda i: (0, i, 0)),
            pl.BlockSpec((BR, 128), lambda i: (i, 0)),
        ],
        out_specs=[
            pl.BlockSpec((BR, 128), lambda i: (i, 0)),
            pl.BlockSpec((BR, 1), lambda i: (i, 0)),
        ],
        out_shape=[
            jax.ShapeDtypeStruct((N, 128), jnp.float32),
            jax.ShapeDtypeStruct((N, 1), jnp.float32),
        ],
    )(degp, x)


def _tc_mlp(z, p, dinv2d, W1, b1, Wc):
    def body(z_ref, p_ref, dinv_ref, W1_ref, b1_ref, Wc_ref, z2_ref):
        dinv = dinv_ref[...]
        agg1 = (z_ref[...] + p_ref[0] + p_ref[1]) * dinv
        h = jnp.maximum(
            jnp.dot(agg1, W1_ref[...], preferred_element_type=jnp.float32)
            + b1_ref[...],
            0.0,
        )
        g = jnp.dot(h, Wc_ref[...], preferred_element_type=jnp.float32)
        z2_ref[...] = jnp.concatenate([g * dinv, jnp.zeros_like(g)], axis=1)

    return pl.pallas_call(
        body,
        grid=(N // BR,),
        in_specs=[
            pl.BlockSpec((BR, 128), lambda i: (i, 0)),
            pl.BlockSpec((2, BR, 128), lambda i: (0, i, 0)),
            pl.BlockSpec((BR, 1), lambda i: (i, 0)),
            pl.BlockSpec((128, 256), lambda i: (0, 0)),
            pl.BlockSpec((1, 256), lambda i: (0, 0)),
            pl.BlockSpec((256, 64), lambda i: (0, 0)),
        ],
        out_specs=pl.BlockSpec((BR, 128), lambda i: (i, 0)),
        out_shape=jax.ShapeDtypeStruct((N, 128), jnp.float32),
    )(z, p, dinv2d, W1, b1, Wc)


def _tc_out(z2, q, dinv2d, bmu, bls):
    def body(z2_ref, q_ref, dinv_ref, bmu_ref, bls_ref, mu_ref, ls_ref):
        out = (z2_ref[...] + q_ref[0] + q_ref[1])[:, :64] * dinv_ref[...]
        mu_ref[...] = out[:, :32] + bmu_ref[...]
        ls_ref[...] = out[:, 32:] + bls_ref[...]

    return pl.pallas_call(
        body,
        grid=(N // BR,),
        in_specs=[
            pl.BlockSpec((BR, 128), lambda i: (i, 0)),
            pl.BlockSpec((2, BR, 128), lambda i: (0, i, 0)),
            pl.BlockSpec((BR, 1), lambda i: (i, 0)),
            pl.BlockSpec((1, 32), lambda i: (0, 0)),
            pl.BlockSpec((1, 32), lambda i: (0, 0)),
        ],
        out_specs=[
            pl.BlockSpec((BR, 32), lambda i: (i, 0)),
            pl.BlockSpec((BR, 32), lambda i: (i, 0)),
        ],
        out_shape=[
            jax.ShapeDtypeStruct((N, 32), jnp.float32),
            jax.ShapeDtypeStruct((N, 32), jnp.float32),
        ],
    )(z2, q, dinv2d, bmu, bls)


def kernel(x, edge_index, W1, b1, Wmu, bmu, Wls, bls):
    src2d = jnp.concatenate(
        [edge_index[0], jnp.zeros((EPAD - E,), jnp.int32)]
    ).reshape(ECH, CHUNK)
    dst2d = jnp.concatenate(
        [edge_index[1], jnp.full((EPAD - E,), PAD_DST, jnp.int32)]
    ).reshape(ECH, CHUNK)

    degp = _deg_kernel(dst2d)
    z, dinv2d = _tc_scale(degp, x)
    p = _prop128(src2d, dst2d, z)
    Wc = jnp.concatenate([Wmu, Wls], axis=1)
    z2 = _tc_mlp(z, p, dinv2d, W1, b1.reshape(1, -1), Wc)
    q = _prop128(src2d, dst2d, z2)
    mu, ls = _tc_out(z2, q, dinv2d, bmu.reshape(1, -1), bls.reshape(1, -1))
    return (mu, ls)
